# baseline (speedup 1.0000x reference)
import jax
import jax.numpy as jnp
from jax import lax
from jax.experimental import pallas as pl
from jax.experimental.pallas import tpu as pltpu
from jax.experimental.pallas import tpu_sc as plsc

B = 16384
F = 26
V = 100000
VG = V // 4
D = 32
NUM = 13
ED = F * D
EPS = 1e-5

NC = 2
NS = 16
NW = NC * NS
ROWS_W = B // NW
RB = 128
NRB = ROWS_W // RB
NBUF = 4


def _sc_gather_body(tab4, xt, outg, idx_v, g_v, rows_v, gsem):
    wid = lax.axis_index("s") * NC + lax.axis_index("c")
    b_base = wid * ROWS_W
    pltpu.sync_copy(xt.at[:, pl.ds(b_base, ROWS_W)], idx_v)

    def body(rb, carry):
        row = b_base + rb * RB
        for f in range(F):
            b = f % NBUF
            if f >= NBUF:
                fp = f - NBUF
                pltpu.make_async_copy(tab4.at[fp].at[g_v.at[b]],
                                      rows_v.at[b], gsem).wait()
                pltpu.sync_copy(rows_v.at[b], outg.at[fp, pl.ds(row, RB), :])
            for v in range(RB // 16):
                x = idx_v[f, pl.ds(rb * RB + v * 16, 16)]
                g_v[b, pl.ds(v * 16, 16)] = lax.shift_right_logical(x, 2)
            pltpu.async_copy(tab4.at[f].at[g_v.at[b]], rows_v.at[b], gsem)
        for f in range(F - NBUF, F):
            b = f % NBUF
            pltpu.make_async_copy(tab4.at[f].at[g_v.at[b]],
                                  rows_v.at[b], gsem).wait()
            pltpu.sync_copy(rows_v.at[b], outg.at[f, pl.ds(row, RB), :])
        return carry

    lax.fori_loop(0, NRB, body, 0)


_SC_GATHER_CACHE = []


def _sc_gather(tab4, xt):
    if not _SC_GATHER_CACHE:
        _SC_GATHER_CACHE.append(pl.kernel(
            _sc_gather_body,
            out_type=jax.ShapeDtypeStruct((F, B, 128), jnp.float32),
            mesh=plsc.VectorSubcoreMesh(core_axis_name="c", subcore_axis_name="s"),
            scratch_types=[
                pltpu.VMEM((F, ROWS_W), jnp.int32),
                pltpu.VMEM((NBUF, RB), jnp.int32),
                pltpu.VMEM((NBUF, RB, 128), jnp.float32),
                pltpu.SemaphoreType.DMA,
            ],
        ))
    return _SC_GATHER_CACHE[0](tab4, xt)


BB = 512
_INV_STD = (1.0 + EPS) ** -0.5


def _mlp_body(xg, xc, xn, w1r, w1n, b1, g1, be1, w2, b2, g2, be2, w3, b3, out):
    colj = lax.broadcasted_iota(jnp.int32, (BB, 128), 1) // D
    h = jnp.dot(xn[...], w1n[...], preferred_element_type=jnp.float32)
    for f in range(F):
        s = lax.bitwise_and(xc[:, f], 3)
        m = (colj == s[:, None]).astype(jnp.float32)
        h = h + jnp.dot(xg[f] * m, w1r[f],
                        preferred_element_type=jnp.float32)
    h = jnp.maximum(h + b1[...], 0.0)
    h = h * (g1[...] * _INV_STD) + be1[...]
    h = jnp.maximum(jnp.dot(h, w2[...], preferred_element_type=jnp.float32) + b2[...], 0.0)
    h = h * (g2[...] * _INV_STD) + be2[...]
    out[...] = jnp.dot(h, w3[...], preferred_element_type=jnp.float32) + b3[...]


_mlp = pl.pallas_call(
    _mlp_body,
    grid=(B // BB,),
    in_specs=[
        pl.BlockSpec((F, BB, 128), lambda i: (0, i, 0)),
        pl.BlockSpec((BB, F), lambda i: (i, 0)),
        pl.BlockSpec((BB, NUM), lambda i: (i, 0)),
        pl.BlockSpec((F, 128, 128), lambda i: (0, 0, 0)),
        pl.BlockSpec((NUM, 128), lambda i: (0, 0)),
        pl.BlockSpec((1, 128), lambda i: (0, 0)),
        pl.BlockSpec((1, 128), lambda i: (0, 0)),
        pl.BlockSpec((1, 128), lambda i: (0, 0)),
        pl.BlockSpec((128, 64), lambda i: (0, 0)),
        pl.BlockSpec((1, 64), lambda i: (0, 0)),
        pl.BlockSpec((1, 64), lambda i: (0, 0)),
        pl.BlockSpec((1, 64), lambda i: (0, 0)),
        pl.BlockSpec((64, 1), lambda i: (0, 0)),
        pl.BlockSpec((1, 1), lambda i: (0, 0)),
    ],
    out_specs=pl.BlockSpec((BB, 1), lambda i: (i, 0)),
    out_shape=jax.ShapeDtypeStruct((B, 1), jnp.float32),
)


def kernel(x_cat, x_num, tables, W1, b1, g1, be1, W2, b2, g2, be2, W3, b3):
    xt = x_cat.T
    tab4 = tables.reshape(F, VG, 128)
    xg = _sc_gather(tab4, xt)

    w1r = jnp.tile(W1[:ED].reshape(F, 1, D, 128), (1, 4, 1, 1)).reshape(F, 128, 128)
    return _mlp(
        xg, x_cat, x_num, w1r, W1[ED:],
        b1.reshape(1, 128), g1.reshape(1, 128), be1.reshape(1, 128),
        W2, b2.reshape(1, 64), g2.reshape(1, 64), be2.reshape(1, 64),
        W3, b3.reshape(1, 1),
    )


# speedup vs baseline: 1.1312x; 1.1312x over previous
import jax
import jax.numpy as jnp
from jax import lax
from jax.experimental import pallas as pl
from jax.experimental.pallas import tpu as pltpu
from jax.experimental.pallas import tpu_sc as plsc

B = 16384
F = 26
V = 100000
D = 32
NUM = 13
ED = F * D
NT = 7
EPS = 1e-5

NC = 2
NS = 16
NW = NC * NS
ROWS_W = B // NW
RB = 128
NRB = ROWS_W // RB
CHUNKS_W = NRB * F
NBUF = 4


def _sc_gather_body(tab3, xt, out3, idx_v, rows_v, gsem):
    wid = lax.axis_index("s") * NC + lax.axis_index("c")
    b_base = wid * ROWS_W
    pltpu.sync_copy(xt.at[:, pl.ds(b_base, ROWS_W)], idx_v)

    def outer(co, carry):
        c0 = co * NBUF
        for b in range(NBUF):
            c = c0 + b
            f = c % F
            rb = c // F
            pltpu.async_copy(tab3.at[f].at[idx_v.at[f, pl.ds(rb * RB, RB)]],
                             rows_v.at[b], gsem)
        for b in range(NBUF):
            c = c0 + b
            f = c % F
            rb = c // F
            pltpu.make_async_copy(
                tab3.at[f].at[idx_v.at[f, pl.ds(rb * RB, RB)]],
                rows_v.at[b], gsem).wait()
            ct = f // 4
            k = f % 4
            pltpu.sync_copy(
                rows_v.at[b],
                out3.at[ct, pl.ds(b_base + rb * RB, RB), pl.ds(32 * k, 32)])
        return carry

    lax.fori_loop(0, CHUNKS_W // NBUF, outer, 0)


_SC_GATHER_CACHE = []


def _sc_gather(tables, xt):
    if not _SC_GATHER_CACHE:
        _SC_GATHER_CACHE.append(pl.kernel(
            _sc_gather_body,
            out_type=jax.ShapeDtypeStruct((NT, B, 128), jnp.float32),
            mesh=plsc.VectorSubcoreMesh(core_axis_name="c", subcore_axis_name="s"),
            scratch_types=[
                pltpu.VMEM((F, ROWS_W), jnp.int32),
                pltpu.VMEM((NBUF, RB, D), jnp.float32),
                pltpu.SemaphoreType.DMA,
            ],
            compiler_params=pltpu.CompilerParams(use_tc_tiling_on_sc=False),
        ))
    return _SC_GATHER_CACHE[0](tables, xt)


BB = 1024
_INV_STD = (1.0 + EPS) ** -0.5


def _mlp_body(x3, xn, w13, w1n, b1, g1, be1, w2, b2, g2, be2, w3, b3, out):
    h = jnp.dot(x3[0], w13[0], preferred_element_type=jnp.float32)
    for t in range(1, NT - 1):
        h = h + jnp.dot(x3[t], w13[t], preferred_element_type=jnp.float32)
    col = lax.broadcasted_iota(jnp.int32, (BB, 128), 1)
    x6 = jnp.where(col < 64, x3[NT - 1], 0.0)
    h = h + jnp.dot(x6, w13[NT - 1], preferred_element_type=jnp.float32)
    h = h + jnp.dot(xn[...], w1n[...], preferred_element_type=jnp.float32)
    h = jnp.maximum(h + b1[...], 0.0)
    h = h * (g1[...] * _INV_STD) + be1[...]
    h = jnp.maximum(jnp.dot(h, w2[...], preferred_element_type=jnp.float32) + b2[...], 0.0)
    h = h * (g2[...] * _INV_STD) + be2[...]
    out[...] = jnp.dot(h, w3[...], preferred_element_type=jnp.float32) + b3[...]


_mlp = pl.pallas_call(
    _mlp_body,
    grid=(B // BB,),
    in_specs=[
        pl.BlockSpec((NT, BB, 128), lambda i: (0, i, 0)),
        pl.BlockSpec((BB, NUM), lambda i: (i, 0)),
        pl.BlockSpec((NT, 128, 128), lambda i: (0, 0, 0)),
        pl.BlockSpec((NUM, 128), lambda i: (0, 0)),
        pl.BlockSpec((1, 128), lambda i: (0, 0)),
        pl.BlockSpec((1, 128), lambda i: (0, 0)),
        pl.BlockSpec((1, 128), lambda i: (0, 0)),
        pl.BlockSpec((128, 64), lambda i: (0, 0)),
        pl.BlockSpec((1, 64), lambda i: (0, 0)),
        pl.BlockSpec((1, 64), lambda i: (0, 0)),
        pl.BlockSpec((1, 64), lambda i: (0, 0)),
        pl.BlockSpec((64, 1), lambda i: (0, 0)),
        pl.BlockSpec((1, 1), lambda i: (0, 0)),
    ],
    out_specs=pl.BlockSpec((BB, 1), lambda i: (i, 0)),
    out_shape=jax.ShapeDtypeStruct((B, 1), jnp.float32),
)


def kernel(x_cat, x_num, tables, W1, b1, g1, be1, W2, b2, g2, be2, W3, b3):
    xt = x_cat.T
    x3 = _sc_gather(tables, xt)

    w1p = jnp.concatenate([W1[:ED], jnp.zeros((NT * 128 - ED, 128), W1.dtype)])
    return _mlp(
        x3, x_num, w1p.reshape(NT, 128, 128), W1[ED:],
        b1.reshape(1, 128), g1.reshape(1, 128), be1.reshape(1, 128),
        W2, b2.reshape(1, 64), g2.reshape(1, 64), be2.reshape(1, 64),
        W3, b3.reshape(1, 1),
    )
